# baseline (speedup 1.0000x reference)
import functools

import jax
import jax.numpy as jnp
from jax import lax
from jax.experimental import pallas as pl
from jax.experimental.pallas import tpu as pltpu
from jax.experimental.pallas import tpu_sc as plsc

B, S, E = 4, 2048, 768
NC, NS = 2, 16
NW = NC * NS
S_PER_W = S // NW
CH = 8
CHW = CH * E
N_CHUNKS = S_PER_W // CH
NBUF = 3
LANES = 16


def _sc_body(x_hbm, tab_hbm, out_hbm, t_v, x_v, in_sem, out_sem):
    wid = lax.axis_index("s") * NC + lax.axis_index("c")
    w0 = wid * S_PER_W * E

    def in_copies(c, slot):
        base = w0 + c * CHW
        cps = [pltpu.make_async_copy(
            tab_hbm.at[pl.ds(base, CHW)], t_v.at[slot], in_sem.at[slot])]
        for b in range(B):
            cps.append(pltpu.make_async_copy(
                x_hbm.at[b, pl.ds(base, CHW)], x_v.at[slot, b],
                in_sem.at[slot]))
        return cps

    def out_copies(c, slot):
        base = w0 + c * CHW
        return [pltpu.make_async_copy(
            x_v.at[slot, b], out_hbm.at[b, pl.ds(base, CHW)],
            out_sem.at[slot]) for b in range(B)]

    for cp in in_copies(0, 0):
        cp.start()

    def chunk_body(c, _):
        slot = lax.rem(c, NBUF)

        @pl.when(c + 1 < N_CHUNKS)
        def _prefetch():
            nslot = lax.rem(c + 1, NBUF)

            @pl.when(c >= 2)
            def _drain_prev_out():
                for cp in out_copies(c - 2, nslot):
                    cp.wait()

            for cp in in_copies(c + 1, nslot):
                cp.start()

        for cp in in_copies(c, slot):
            cp.wait()

        for i in range(CHW // LANES):
            off = pl.ds(i * LANES, LANES)
            t = t_v[slot, off]
            for b in range(B):
                plsc.addupdate(x_v.at[slot, b, off], t)

        for cp in out_copies(c, slot):
            cp.start()
        return 0

    lax.fori_loop(0, N_CHUNKS, chunk_body, 0)

    for c in (N_CHUNKS - 3, N_CHUNKS - 2, N_CHUNKS - 1):
        for cp in out_copies(c, c % NBUF):
            cp.wait()


_sc_call = functools.partial(
    pl.kernel,
    out_type=jax.ShapeDtypeStruct((B, S * E), jnp.float32),
    mesh=plsc.VectorSubcoreMesh(core_axis_name="c", subcore_axis_name="s"),
    scratch_types=[
        pltpu.VMEM((NBUF, CHW), jnp.float32),
        pltpu.VMEM((NBUF, B, CHW), jnp.float32),
        pltpu.SemaphoreType.DMA((NBUF,)),
        pltpu.SemaphoreType.DMA((NBUF,)),
    ],
)(_sc_body)


def kernel(x, pos_table):
    b, s, e = x.shape
    out = _sc_call(x.reshape(b, s * e), pos_table.reshape(s * e))
    return out.reshape(b, s, e)


# speedup vs baseline: 1.0452x; 1.0452x over previous
import functools

import jax
import jax.numpy as jnp
from jax import lax
from jax.experimental import pallas as pl
from jax.experimental.pallas import tpu as pltpu
from jax.experimental.pallas import tpu_sc as plsc

B, S, E = 4, 2048, 768
NC, NS = 2, 16
NW = NC * NS
S_PER_W = S // NW
SW = S_PER_W * E
CH = 8
CHW = CH * E
N_CHUNKS = S_PER_W // CH
NBUF = 3
LANES = 16


def _sc_body(x_hbm, tab_hbm, out_hbm, t_v, x_v, t_sem, in_sem, out_sem):
    wid = lax.axis_index("s") * NC + lax.axis_index("c")
    w0 = wid * SW

    def in_copies(c, slot):
        base = w0 + c * CHW
        return [pltpu.make_async_copy(
            x_hbm.at[:, pl.ds(base, CHW)], x_v.at[slot], in_sem.at[slot])]

    def out_copies(c, slot):
        base = w0 + c * CHW
        return [pltpu.make_async_copy(
            x_v.at[slot], out_hbm.at[:, pl.ds(base, CHW)], out_sem.at[slot])]

    t_cp = pltpu.make_async_copy(tab_hbm.at[pl.ds(w0, SW)], t_v, t_sem)
    t_cp.start()
    for cp in in_copies(0, 0):
        cp.start()
    t_cp.wait()

    def chunk_body(c, _):
        slot = lax.rem(c, NBUF)

        @pl.when(c + 1 < N_CHUNKS)
        def _prefetch():
            nslot = lax.rem(c + 1, NBUF)

            @pl.when(c >= 2)
            def _drain_prev_out():
                for cp in out_copies(c - 2, nslot):
                    cp.wait()

            for cp in in_copies(c + 1, nslot):
                cp.start()

        for cp in in_copies(c, slot):
            cp.wait()

        tbase = c * CHW
        for i in range(CHW // LANES):
            t = t_v[pl.ds(tbase + i * LANES, LANES)]
            for b in range(B):
                plsc.addupdate(x_v.at[slot, b, pl.ds(i * LANES, LANES)], t)

        for cp in out_copies(c, slot):
            cp.start()
        return 0

    lax.fori_loop(0, N_CHUNKS, chunk_body, 0)

    for c in (N_CHUNKS - 3, N_CHUNKS - 2, N_CHUNKS - 1):
        for cp in out_copies(c, c % NBUF):
            cp.wait()


_sc_call = functools.partial(
    pl.kernel,
    out_type=jax.ShapeDtypeStruct((B, S * E), jnp.float32),
    mesh=plsc.VectorSubcoreMesh(core_axis_name="c", subcore_axis_name="s"),
    scratch_types=[
        pltpu.VMEM((SW,), jnp.float32),
        pltpu.VMEM((NBUF, B, CHW), jnp.float32),
        pltpu.SemaphoreType.DMA,
        pltpu.SemaphoreType.DMA((NBUF,)),
        pltpu.SemaphoreType.DMA((NBUF,)),
    ],
)(_sc_body)


def kernel(x, pos_table):
    b, s, e = x.shape
    out = _sc_call(x.reshape(b, s * e), pos_table.reshape(s * e))
    return out.reshape(b, s, e)


# speedup vs baseline: 5.4910x; 5.2536x over previous
import jax
import jax.numpy as jnp
from jax.experimental import pallas as pl

BLK_S = 512


def _add_kernel(x_ref, pos_ref, out_ref):
    out_ref[...] = x_ref[...] + pos_ref[...][None, :, :]


def kernel(x, pos_table):
    b, s, e = x.shape
    grid = (s // BLK_S,)
    return pl.pallas_call(
        _add_kernel,
        grid=grid,
        in_specs=[
            pl.BlockSpec((b, BLK_S, e), lambda i: (0, i, 0)),
            pl.BlockSpec((BLK_S, e), lambda i: (i, 0)),
        ],
        out_specs=pl.BlockSpec((b, BLK_S, e), lambda i: (0, i, 0)),
        out_shape=jax.ShapeDtypeStruct((b, s, e), x.dtype),
    )(x, pos_table)


# speedup vs baseline: 5.6857x; 1.0354x over previous
import jax
import jax.numpy as jnp
from jax.experimental import pallas as pl

BLK_S = 1024


def _add_kernel(x_ref, pos_ref, out_ref):
    out_ref[...] = x_ref[...] + pos_ref[...][None, :, :]


def kernel(x, pos_table):
    b, s, e = x.shape
    grid = (s // BLK_S,)
    return pl.pallas_call(
        _add_kernel,
        grid=grid,
        in_specs=[
            pl.BlockSpec((b, BLK_S, e), lambda i: (0, i, 0)),
            pl.BlockSpec((BLK_S, e), lambda i: (i, 0)),
        ],
        out_specs=pl.BlockSpec((b, BLK_S, e), lambda i: (0, i, 0)),
        out_shape=jax.ShapeDtypeStruct((b, s, e), x.dtype),
    )(x, pos_table)
